# baseline (speedup 1.0000x reference)
import jax
import jax.numpy as jnp
from jax import lax
from jax.experimental import pallas as pl
from jax.experimental.pallas import tpu as pltpu
from jax.experimental.pallas import tpu_sc as plsc

N = 10000
HEADS = 8
HID = 16
D = HEADS * HID
HD = D // 2
E = 320000
NEG = 0.2

NC = 2
NS = 16
NW = NC * NS
EPT = E // NW
CH = 100
NCH = EPT // CH
AW = 80
NPAD = 10240
RPT = NPAD // NS
RZ = 128


def _pre_body(xa, xp, wfa, bfa, wfp, bfp,
              ws_ap, as_ap, ad_ap,
              ws_pa, as_pa, ad_pa,
              ws_pp, as_pp, ad_pp,
              hl_ap, hh_ap, acs_ap, acd_ap,
              hl_pa, hh_pa, acs_pa, acd_pa,
              hl_pp, hh_pp, acs_pp, acd_pp):
    f32 = jnp.float32
    ha = jnp.dot(xa[...], wfa[...], preferred_element_type=f32) + bfa[0]
    hp = jnp.dot(xp[...], wfp[...], preferred_element_type=f32) + bfp[0]

    def et(hsrc, hdst, ws, a_s, a_d, hl_o, hh_o, acs_o, acd_o):
        hs = jnp.dot(hsrc, ws[...], preferred_element_type=f32)
        hl_o[...] = hs[:, :HD]
        hh_o[...] = hs[:, HD:]
        acs_o[...] = jnp.dot(hsrc, a_s[...], preferred_element_type=f32)
        acd_o[...] = jnp.dot(hdst, a_d[...], preferred_element_type=f32)

    et(ha, hp, ws_ap, as_ap, ad_ap, hl_ap, hh_ap, acs_ap, acd_ap)
    et(hp, ha, ws_pa, as_pa, ad_pa, hl_pa, hh_pa, acs_pa, acd_pa)
    et(hp, hp, ws_pp, as_pp, ad_pp, hl_pp, hh_pp, acs_pp, acd_pp)


def _sc_body(hl_ap, hh_ap, acs_ap, acd_ap, se_ap, de_ap,
             hl_pa, hh_pa, acs_pa, acd_pa, se_pa, de_pa,
             hl_pp, hh_pp, acs_pp, acd_pp, se_pp, de_pp,
             ol_ap, oh_ap, ol_pa, oh_pa, ol_pp, oh_pp,
             idx_s, idx_d, ar_s0, ar_d0, rw0, ar_s1, ar_d1, rw1,
             msg, zb_o, out_sh, gsem0, gsem1, ssem):
    c = lax.axis_index("c")
    s = lax.axis_index("s")
    wid = c * NS + s
    f32 = jnp.float32
    zv = jnp.zeros((16,), f32)
    bufs = ((ar_s0, ar_d0, rw0, gsem0), (ar_s1, ar_d1, rw1, gsem1))

    def zrow_o(r, carry):
        for h in range(AW // 16):
            zb_o[r, pl.ds(h * 16, 16)] = zv
        return carry
    lax.fori_loop(0, RZ, zrow_o, 0)

    for (hs_hs, acs_h, acd_h, se_h, de_h, op_hs) in (
            ((hl_ap, hh_ap), acs_ap, acd_ap, se_ap, de_ap, (ol_ap, oh_ap)),
            ((hl_pa, hh_pa), acs_pa, acd_pa, se_pa, de_pa, (ol_pa, oh_pa)),
            ((hl_pp, hh_pp), acs_pp, acd_pp, se_pp, de_pp, (ol_pp, oh_pp))):
        pltpu.sync_copy(se_h.at[wid], idx_s)
        pltpu.sync_copy(de_h.at[wid], idx_d)

        for half in (0, 1):
            hs_h = hs_hs[half]
            op_h = op_hs[half]
            for jz in range(RPT // RZ):
                base = s * RPT + jz * RZ
                pltpu.sync_copy(zb_o, out_sh.at[pl.ds(base, RZ)])
            plsc.subcore_barrier()

            if half == 1:
                def zmsg(e, carry):
                    msg[e, pl.ds(HD, 16)] = zv
                    return carry
                lax.fori_loop(0, CH, zmsg, 0)

            def issue_gathers(j, b):
                ar_s, ar_d, rw, gsem = bufs[b]
                pltpu.async_copy(acs_h.at[idx_s.at[j]], ar_s, gsem)
                pltpu.async_copy(acd_h.at[idx_d.at[j]], ar_d, gsem)
                pltpu.async_copy(hs_h.at[idx_s.at[j]], rw, gsem)

            def wait_gathers(j, b):
                ar_s, ar_d, rw, gsem = bufs[b]
                pltpu.make_async_copy(acs_h.at[idx_s.at[j]], ar_s, gsem).wait()
                pltpu.make_async_copy(acd_h.at[idx_d.at[j]], ar_d, gsem).wait()
                pltpu.make_async_copy(hs_h.at[idx_s.at[j]], rw, gsem).wait()

            def process(j, b, wait_scatter):
                ar_s, ar_d, rw, gsem = bufs[b]
                di = idx_d.at[j]
                wait_gathers(j, b)
                jn = jnp.minimum(j + 1, NCH - 1)
                issue_gathers(jn, 1 - b)
                if wait_scatter is None:
                    pltpu.make_async_copy(msg, out_sh.at[di], ssem).wait()
                else:
                    @pl.when(wait_scatter)
                    def _():
                        pltpu.make_async_copy(
                            msg, out_sh.at[di], ssem).wait()

                def ecomp(e, carry):
                    x = ar_s[e] + ar_d[e]
                    w = jnp.exp(jnp.maximum(x, NEG * x))
                    for h in range(HD // HID):
                        msg[e, pl.ds(h * HID, HID)] = (
                            rw[e, pl.ds(h * HID, HID)]
                            * w[half * (HD // HID) + h])
                    if half == 0:
                        msg[e, pl.ds(HD, 16)] = w
                    return carry
                lax.fori_loop(0, CH, ecomp, 0)

                pltpu.async_copy(msg, out_sh.at[di], ssem, add=True)

            issue_gathers(0, 0)

            def pair(i, carry):
                process(2 * i, 0, i > 0)
                process(2 * i + 1, 1, None)
                return carry
            lax.fori_loop(0, NCH // 2, pair, 0)

            wait_gathers(NCH - 1, 0)
            pltpu.make_async_copy(
                msg, out_sh.at[idx_d.at[NCH - 1]], ssem).wait()

            plsc.subcore_barrier()
            pltpu.sync_copy(out_sh.at[pl.ds(s * RPT, RPT)],
                            op_h.at[c, pl.ds(s * RPT, RPT)])
            plsc.subcore_barrier()


_sc_mesh = plsc.VectorSubcoreMesh(
    core_axis_name="c", subcore_axis_name="s", num_cores=NC, num_subcores=NS)

_sc_fn = pl.kernel(
    _sc_body,
    out_type=[jax.ShapeDtypeStruct((NC, NPAD, AW), jnp.float32)] * 6,
    mesh=_sc_mesh,
    compiler_params=pltpu.CompilerParams(use_tc_tiling_on_sc=False),
    scratch_types=[
        pltpu.VMEM((NCH, CH), jnp.int32),
        pltpu.VMEM((NCH, CH), jnp.int32),
        pltpu.VMEM((CH, 16), jnp.float32),
        pltpu.VMEM((CH, 16), jnp.float32),
        pltpu.VMEM((CH, HD), jnp.float32),
        pltpu.VMEM((CH, 16), jnp.float32),
        pltpu.VMEM((CH, 16), jnp.float32),
        pltpu.VMEM((CH, HD), jnp.float32),
        pltpu.VMEM((CH, AW), jnp.float32),
        pltpu.VMEM((RZ, AW), jnp.float32),
        pltpu.VMEM_SHARED((NPAD, AW), jnp.float32),
        pltpu.SemaphoreType.DMA,
        pltpu.SemaphoreType.DMA,
        pltpu.SemaphoreType.DMA,
    ],
)


_BLK = 1280


def _post_body(ol_ap, oh_ap, ol_pa, oh_pa, ol_pp, oh_pp,
               b_ap, b_pa, b_pp, wk, bk,
               oa_ref, gap_ref, gpp_ref, ks_ref):
    i = pl.program_id(0)
    f32 = jnp.float32

    def gat(ol, oh, b):
        lo = ol[0] + ol[1]
        hi = oh[0] + oh[1]
        o = jnp.concatenate([lo[:, :HD], hi[:, :HD]], axis=1)
        d = lo[:, HD:HD + HEADS]
        rj = lax.broadcasted_iota(jnp.int32, (HEADS, D), 0)
        rl = lax.broadcasted_iota(jnp.int32, (HEADS, D), 1)
        p = (rl // HID == rj).astype(f32)
        dex = jnp.dot(d, p, preferred_element_type=f32)
        return o / jnp.maximum(dex, 1e-16) + b[0]

    g_pa = gat(ol_pa, oh_pa, b_pa)
    oa_ref[...] = g_pa
    g_ap = gat(ol_ap, oh_ap, b_ap)
    gap_ref[...] = g_ap
    g_pp = gat(ol_pp, oh_pp, b_pp)
    gpp_ref[...] = g_pp

    row = (i * _BLK
           + lax.broadcasted_iota(jnp.int32, (_BLK, 1), 0))
    valid = (row < N).astype(f32)
    k_ap = jnp.sum(valid * jnp.tanh(
        jnp.dot(g_ap, wk[...], preferred_element_type=f32) + bk[0]),
        axis=0, keepdims=True)
    k_pp = jnp.sum(valid * jnp.tanh(
        jnp.dot(g_pp, wk[...], preferred_element_type=f32) + bk[0]),
        axis=0, keepdims=True)
    kb = jnp.concatenate([k_ap, k_pp], axis=0)

    @pl.when(i == 0)
    def _():
        ks_ref[...] = kb

    @pl.when(i != 0)
    def _():
        ks_ref[...] = ks_ref[...] + kb


def _fin_body(gap, gpp, ks, q, op_ref):
    kq = ks[...] * q[...]
    a0 = jnp.sum(kq[0]) / N
    a1 = jnp.sum(kq[1]) / N
    m = jnp.maximum(a0, a1)
    e0 = jnp.exp(a0 - m)
    e1 = jnp.exp(a1 - m)
    s0 = e0 / (e0 + e1)
    s1 = e1 / (e0 + e1)
    op_ref[...] = s0 * gap[...] + s1 * gpp[...]


def kernel(x_author, x_paper, edge_ap, edge_pa, edge_pp,
           W_fc_author, b_fc_author, W_fc_paper, b_fc_paper,
           W_src_ap, W_dst_ap, att_src_ap, att_dst_ap, bias_ap,
           W_src_pa, W_dst_pa, att_src_pa, att_dst_pa, bias_pa,
           W_src_pp, W_dst_pp, att_src_pp, att_dst_pp, bias_pp,
           q, W_k, b_k):
    f32 = jnp.float32

    def fold(w_src, att_src, w_dst, att_dst):
        a_s = (w_src.reshape(HID, HEADS, HID)
               * att_src.reshape(1, HEADS, HID)).sum(-1)
        a_d = (w_dst.reshape(HID, HEADS, HID)
               * att_dst.reshape(1, HEADS, HID)).sum(-1)
        z = jnp.zeros((HID, HEADS), f32)
        return (jnp.concatenate([a_s, z], axis=1),
                jnp.concatenate([a_d, z], axis=1))

    as_ap, ad_ap = fold(W_src_ap, att_src_ap, W_dst_ap, att_dst_ap)
    as_pa, ad_pa = fold(W_src_pa, att_src_pa, W_dst_pa, att_dst_pa)
    as_pp, ad_pp = fold(W_src_pp, att_src_pp, W_dst_pp, att_dst_pp)

    pb = 2000
    pre_out = pl.pallas_call(
        _pre_body,
        grid=(N // pb,),
        in_specs=[
            pl.BlockSpec((pb, D), lambda i: (i, 0)),
            pl.BlockSpec((pb, D), lambda i: (i, 0)),
            pl.BlockSpec((D, HID), lambda i: (0, 0)),
            pl.BlockSpec((1, HID), lambda i: (0, 0)),
            pl.BlockSpec((D, HID), lambda i: (0, 0)),
            pl.BlockSpec((1, HID), lambda i: (0, 0)),
        ] + [
            pl.BlockSpec((HID, D), lambda i: (0, 0)),
            pl.BlockSpec((HID, HID), lambda i: (0, 0)),
            pl.BlockSpec((HID, HID), lambda i: (0, 0)),
        ] * 3,
        out_specs=[
            pl.BlockSpec((pb, HD), lambda i: (i, 0)),
            pl.BlockSpec((pb, HD), lambda i: (i, 0)),
            pl.BlockSpec((pb, 16), lambda i: (i, 0)),
            pl.BlockSpec((pb, 16), lambda i: (i, 0)),
        ] * 3,
        out_shape=[
            jax.ShapeDtypeStruct((N, HD), f32),
            jax.ShapeDtypeStruct((N, HD), f32),
            jax.ShapeDtypeStruct((N, 16), f32),
            jax.ShapeDtypeStruct((N, 16), f32),
        ] * 3,
    )(x_author, x_paper,
      W_fc_author, b_fc_author.reshape(1, HID),
      W_fc_paper, b_fc_paper.reshape(1, HID),
      W_src_ap, as_ap, ad_ap,
      W_src_pa, as_pa, ad_pa,
      W_src_pp, as_pp, ad_pp)
    (hl_ap, hh_ap, acs_ap, acd_ap,
     hl_pa, hh_pa, acs_pa, acd_pa,
     hl_pp, hh_pp, acs_pp, acd_pp) = pre_out

    se_ap = edge_ap[0].reshape(NW, NCH, CH)
    de_ap = edge_ap[1].reshape(NW, NCH, CH)
    se_pa = edge_pa[0].reshape(NW, NCH, CH)
    de_pa = edge_pa[1].reshape(NW, NCH, CH)
    se_pp = edge_pp[0].reshape(NW, NCH, CH)
    de_pp = edge_pp[1].reshape(NW, NCH, CH)

    (ol_ap, oh_ap, ol_pa, oh_pa, ol_pp, oh_pp) = _sc_fn(
        hl_ap, hh_ap, acs_ap, acd_ap, se_ap, de_ap,
        hl_pa, hh_pa, acs_pa, acd_pa, se_pa, de_pa,
        hl_pp, hh_pp, acs_pp, acd_pp, se_pp, de_pp)

    grid = NPAD // _BLK
    oa, gap, gpp, ks = pl.pallas_call(
        _post_body,
        grid=(grid,),
        in_specs=[
            pl.BlockSpec((NC, _BLK, AW), lambda i: (0, i, 0)),
            pl.BlockSpec((NC, _BLK, AW), lambda i: (0, i, 0)),
            pl.BlockSpec((NC, _BLK, AW), lambda i: (0, i, 0)),
            pl.BlockSpec((NC, _BLK, AW), lambda i: (0, i, 0)),
            pl.BlockSpec((NC, _BLK, AW), lambda i: (0, i, 0)),
            pl.BlockSpec((NC, _BLK, AW), lambda i: (0, i, 0)),
            pl.BlockSpec((1, D), lambda i: (0, 0)),
            pl.BlockSpec((1, D), lambda i: (0, 0)),
            pl.BlockSpec((1, D), lambda i: (0, 0)),
            pl.BlockSpec((D, D), lambda i: (0, 0)),
            pl.BlockSpec((1, D), lambda i: (0, 0)),
        ],
        out_specs=[
            pl.BlockSpec((_BLK, D), lambda i: (i, 0)),
            pl.BlockSpec((_BLK, D), lambda i: (i, 0)),
            pl.BlockSpec((_BLK, D), lambda i: (i, 0)),
            pl.BlockSpec((2, D), lambda i: (0, 0)),
        ],
        out_shape=[
            jax.ShapeDtypeStruct((N, D), f32),
            jax.ShapeDtypeStruct((N, D), f32),
            jax.ShapeDtypeStruct((N, D), f32),
            jax.ShapeDtypeStruct((2, D), f32),
        ],
    )(ol_ap, oh_ap, ol_pa, oh_pa, ol_pp, oh_pp,
      bias_ap.reshape(1, D), bias_pa.reshape(1, D), bias_pp.reshape(1, D),
      W_k, b_k.reshape(1, D))

    out_paper = pl.pallas_call(
        _fin_body,
        grid=(grid,),
        in_specs=[
            pl.BlockSpec((_BLK, D), lambda i: (i, 0)),
            pl.BlockSpec((_BLK, D), lambda i: (i, 0)),
            pl.BlockSpec((2, D), lambda i: (0, 0)),
            pl.BlockSpec((1, D), lambda i: (0, 0)),
        ],
        out_specs=pl.BlockSpec((_BLK, D), lambda i: (i, 0)),
        out_shape=jax.ShapeDtypeStruct((N, D), f32),
    )(gap, gpp, ks, q)

    return (oa, out_paper)


# speedup vs baseline: 1.0920x; 1.0920x over previous
import jax
import jax.numpy as jnp
from jax import lax
from jax.experimental import pallas as pl
from jax.experimental.pallas import tpu as pltpu
from jax.experimental.pallas import tpu_sc as plsc

N = 10000
HEADS = 8
HID = 16
D = HEADS * HID
HD = D // 2
E = 320000
NEG = 0.2

NC = 2
NS = 16
NW = NC * NS
EPT = E // NW
CH = 100
NCH = EPT // CH
AW = 80
NPAD = 10240
RPT = NPAD // NS
RZ = 128


def _pre_body(xa, xp, wfa, bfa, wfp, bfp,
              ws_ap, as_ap, ad_ap,
              ws_pa, as_pa, ad_pa,
              ws_pp, as_pp, ad_pp,
              hl_ap, hh_ap, acs_ap, acd_ap,
              hl_pa, hh_pa, acs_pa, acd_pa,
              hl_pp, hh_pp, acs_pp, acd_pp):
    f32 = jnp.float32
    ha = jnp.dot(xa[...], wfa[...], preferred_element_type=f32) + bfa[0]
    hp = jnp.dot(xp[...], wfp[...], preferred_element_type=f32) + bfp[0]

    def et(hsrc, hdst, ws, a_s, a_d, hl_o, hh_o, acs_o, acd_o):
        hs = jnp.dot(hsrc, ws[...], preferred_element_type=f32)
        hl_o[...] = hs[:, :HD]
        hh_o[...] = hs[:, HD:]
        acs_o[...] = jnp.dot(hsrc, a_s[...], preferred_element_type=f32)
        acd_o[...] = jnp.dot(hdst, a_d[...], preferred_element_type=f32)

    et(ha, hp, ws_ap, as_ap, ad_ap, hl_ap, hh_ap, acs_ap, acd_ap)
    et(hp, ha, ws_pa, as_pa, ad_pa, hl_pa, hh_pa, acs_pa, acd_pa)
    et(hp, hp, ws_pp, as_pp, ad_pp, hl_pp, hh_pp, acs_pp, acd_pp)


def _sc_body(hl_ap, hh_ap, acs_ap, acd_ap, se_ap, de_ap,
             hl_pa, hh_pa, acs_pa, acd_pa, se_pa, de_pa,
             hl_pp, hh_pp, acs_pp, acd_pp, se_pp, de_pp,
             ol_ap, oh_ap, ol_pa, oh_pa, ol_pp, oh_pp,
             idx_s, idx_d, ar_s0, ar_d0, rw0, ar_s1, ar_d1, rw1,
             msg0, msg1, zb_o, out_sh, gsem0, gsem1, ssem0, ssem1):
    c = lax.axis_index("c")
    s = lax.axis_index("s")
    wid = c * NS + s
    f32 = jnp.float32
    zv = jnp.zeros((16,), f32)
    bufs = ((ar_s0, ar_d0, rw0, gsem0), (ar_s1, ar_d1, rw1, gsem1))
    msgs = (msg0, msg1)
    ssems = (ssem0, ssem1)

    def zrow_o(r, carry):
        for h in range(AW // 16):
            zb_o[r, pl.ds(h * 16, 16)] = zv
        return carry
    lax.fori_loop(0, RZ, zrow_o, 0)

    for (hs_hs, acs_h, acd_h, se_h, de_h, op_hs) in (
            ((hl_ap, hh_ap), acs_ap, acd_ap, se_ap, de_ap, (ol_ap, oh_ap)),
            ((hl_pa, hh_pa), acs_pa, acd_pa, se_pa, de_pa, (ol_pa, oh_pa)),
            ((hl_pp, hh_pp), acs_pp, acd_pp, se_pp, de_pp, (ol_pp, oh_pp))):
        pltpu.sync_copy(se_h.at[wid], idx_s)
        pltpu.sync_copy(de_h.at[wid], idx_d)

        for half in (0, 1):
            hs_h = hs_hs[half]
            op_h = op_hs[half]
            for jz in range(RPT // RZ):
                base = s * RPT + jz * RZ
                pltpu.sync_copy(zb_o, out_sh.at[pl.ds(base, RZ)])
            plsc.subcore_barrier()

            if half == 1:
                def zmsg(e, carry):
                    msg0[e, pl.ds(HD, 16)] = zv
                    msg1[e, pl.ds(HD, 16)] = zv
                    return carry
                lax.fori_loop(0, CH, zmsg, 0)

            def issue_gathers(j, b):
                ar_s, ar_d, rw, gsem = bufs[b]
                pltpu.async_copy(acs_h.at[idx_s.at[j]], ar_s, gsem)
                pltpu.async_copy(acd_h.at[idx_d.at[j]], ar_d, gsem)
                pltpu.async_copy(hs_h.at[idx_s.at[j]], rw, gsem)

            def wait_gathers(j, b):
                ar_s, ar_d, rw, gsem = bufs[b]
                pltpu.make_async_copy(acs_h.at[idx_s.at[j]], ar_s, gsem).wait()
                pltpu.make_async_copy(acd_h.at[idx_d.at[j]], ar_d, gsem).wait()
                pltpu.make_async_copy(hs_h.at[idx_s.at[j]], rw, gsem).wait()

            def process(j, b, wait_scatter):
                ar_s, ar_d, rw, gsem = bufs[b]
                msg = msgs[b]
                ssem = ssems[b]
                di = idx_d.at[j]
                wait_gathers(j, b)
                jn = jnp.minimum(j + 1, NCH - 1)
                issue_gathers(jn, 1 - b)

                @pl.when(wait_scatter)
                def _():
                    pltpu.make_async_copy(msg, out_sh.at[di], ssem).wait()

                def ecomp(e, carry):
                    x = ar_s[e] + ar_d[e]
                    w = jnp.exp(jnp.maximum(x, NEG * x))
                    for h in range(HD // HID):
                        msg[e, pl.ds(h * HID, HID)] = (
                            rw[e, pl.ds(h * HID, HID)]
                            * w[half * (HD // HID) + h])
                    if half == 0:
                        msg[e, pl.ds(HD, 16)] = w
                    return carry
                lax.fori_loop(0, CH, ecomp, 0)

                pltpu.async_copy(msg, out_sh.at[di], ssem, add=True)

            issue_gathers(0, 0)

            def pair(i, carry):
                process(2 * i, 0, i > 0)
                process(2 * i + 1, 1, i > 0)
                return carry
            lax.fori_loop(0, NCH // 2, pair, 0)

            wait_gathers(NCH - 1, 0)
            pltpu.make_async_copy(
                msg0, out_sh.at[idx_d.at[NCH - 2]], ssem0).wait()
            pltpu.make_async_copy(
                msg1, out_sh.at[idx_d.at[NCH - 1]], ssem1).wait()

            plsc.subcore_barrier()
            pltpu.sync_copy(out_sh.at[pl.ds(s * RPT, RPT)],
                            op_h.at[c, pl.ds(s * RPT, RPT)])
            plsc.subcore_barrier()


_sc_mesh = plsc.VectorSubcoreMesh(
    core_axis_name="c", subcore_axis_name="s", num_cores=NC, num_subcores=NS)

_sc_fn = pl.kernel(
    _sc_body,
    out_type=[jax.ShapeDtypeStruct((NC, NPAD, AW), jnp.float32)] * 6,
    mesh=_sc_mesh,
    compiler_params=pltpu.CompilerParams(use_tc_tiling_on_sc=False),
    scratch_types=[
        pltpu.VMEM((NCH, CH), jnp.int32),
        pltpu.VMEM((NCH, CH), jnp.int32),
        pltpu.VMEM((CH, 16), jnp.float32),
        pltpu.VMEM((CH, 16), jnp.float32),
        pltpu.VMEM((CH, HD), jnp.float32),
        pltpu.VMEM((CH, 16), jnp.float32),
        pltpu.VMEM((CH, 16), jnp.float32),
        pltpu.VMEM((CH, HD), jnp.float32),
        pltpu.VMEM((CH, AW), jnp.float32),
        pltpu.VMEM((CH, AW), jnp.float32),
        pltpu.VMEM((RZ, AW), jnp.float32),
        pltpu.VMEM_SHARED((NPAD, AW), jnp.float32),
        pltpu.SemaphoreType.DMA,
        pltpu.SemaphoreType.DMA,
        pltpu.SemaphoreType.DMA,
        pltpu.SemaphoreType.DMA,
    ],
)


_BLK = 1280


def _post_body(ol_ap, oh_ap, ol_pa, oh_pa, ol_pp, oh_pp,
               b_ap, b_pa, b_pp, wk, bk,
               oa_ref, gap_ref, gpp_ref, ks_ref):
    i = pl.program_id(0)
    f32 = jnp.float32

    def gat(ol, oh, b):
        lo = ol[0] + ol[1]
        hi = oh[0] + oh[1]
        o = jnp.concatenate([lo[:, :HD], hi[:, :HD]], axis=1)
        d = lo[:, HD:HD + HEADS]
        rj = lax.broadcasted_iota(jnp.int32, (HEADS, D), 0)
        rl = lax.broadcasted_iota(jnp.int32, (HEADS, D), 1)
        p = (rl // HID == rj).astype(f32)
        dex = jnp.dot(d, p, preferred_element_type=f32)
        return o / jnp.maximum(dex, 1e-16) + b[0]

    g_pa = gat(ol_pa, oh_pa, b_pa)
    oa_ref[...] = g_pa
    g_ap = gat(ol_ap, oh_ap, b_ap)
    gap_ref[...] = g_ap
    g_pp = gat(ol_pp, oh_pp, b_pp)
    gpp_ref[...] = g_pp

    row = (i * _BLK
           + lax.broadcasted_iota(jnp.int32, (_BLK, 1), 0))
    valid = (row < N).astype(f32)
    k_ap = jnp.sum(valid * jnp.tanh(
        jnp.dot(g_ap, wk[...], preferred_element_type=f32) + bk[0]),
        axis=0, keepdims=True)
    k_pp = jnp.sum(valid * jnp.tanh(
        jnp.dot(g_pp, wk[...], preferred_element_type=f32) + bk[0]),
        axis=0, keepdims=True)
    kb = jnp.concatenate([k_ap, k_pp], axis=0)

    @pl.when(i == 0)
    def _():
        ks_ref[...] = kb

    @pl.when(i != 0)
    def _():
        ks_ref[...] = ks_ref[...] + kb


def _fin_body(gap, gpp, ks, q, op_ref):
    kq = ks[...] * q[...]
    a0 = jnp.sum(kq[0]) / N
    a1 = jnp.sum(kq[1]) / N
    m = jnp.maximum(a0, a1)
    e0 = jnp.exp(a0 - m)
    e1 = jnp.exp(a1 - m)
    s0 = e0 / (e0 + e1)
    s1 = e1 / (e0 + e1)
    op_ref[...] = s0 * gap[...] + s1 * gpp[...]


def kernel(x_author, x_paper, edge_ap, edge_pa, edge_pp,
           W_fc_author, b_fc_author, W_fc_paper, b_fc_paper,
           W_src_ap, W_dst_ap, att_src_ap, att_dst_ap, bias_ap,
           W_src_pa, W_dst_pa, att_src_pa, att_dst_pa, bias_pa,
           W_src_pp, W_dst_pp, att_src_pp, att_dst_pp, bias_pp,
           q, W_k, b_k):
    f32 = jnp.float32

    def fold(w_src, att_src, w_dst, att_dst):
        a_s = (w_src.reshape(HID, HEADS, HID)
               * att_src.reshape(1, HEADS, HID)).sum(-1)
        a_d = (w_dst.reshape(HID, HEADS, HID)
               * att_dst.reshape(1, HEADS, HID)).sum(-1)
        z = jnp.zeros((HID, HEADS), f32)
        return (jnp.concatenate([a_s, z], axis=1),
                jnp.concatenate([a_d, z], axis=1))

    as_ap, ad_ap = fold(W_src_ap, att_src_ap, W_dst_ap, att_dst_ap)
    as_pa, ad_pa = fold(W_src_pa, att_src_pa, W_dst_pa, att_dst_pa)
    as_pp, ad_pp = fold(W_src_pp, att_src_pp, W_dst_pp, att_dst_pp)

    pb = 2000
    pre_out = pl.pallas_call(
        _pre_body,
        grid=(N // pb,),
        in_specs=[
            pl.BlockSpec((pb, D), lambda i: (i, 0)),
            pl.BlockSpec((pb, D), lambda i: (i, 0)),
            pl.BlockSpec((D, HID), lambda i: (0, 0)),
            pl.BlockSpec((1, HID), lambda i: (0, 0)),
            pl.BlockSpec((D, HID), lambda i: (0, 0)),
            pl.BlockSpec((1, HID), lambda i: (0, 0)),
        ] + [
            pl.BlockSpec((HID, D), lambda i: (0, 0)),
            pl.BlockSpec((HID, HID), lambda i: (0, 0)),
            pl.BlockSpec((HID, HID), lambda i: (0, 0)),
        ] * 3,
        out_specs=[
            pl.BlockSpec((pb, HD), lambda i: (i, 0)),
            pl.BlockSpec((pb, HD), lambda i: (i, 0)),
            pl.BlockSpec((pb, 16), lambda i: (i, 0)),
            pl.BlockSpec((pb, 16), lambda i: (i, 0)),
        ] * 3,
        out_shape=[
            jax.ShapeDtypeStruct((N, HD), f32),
            jax.ShapeDtypeStruct((N, HD), f32),
            jax.ShapeDtypeStruct((N, 16), f32),
            jax.ShapeDtypeStruct((N, 16), f32),
        ] * 3,
    )(x_author, x_paper,
      W_fc_author, b_fc_author.reshape(1, HID),
      W_fc_paper, b_fc_paper.reshape(1, HID),
      W_src_ap, as_ap, ad_ap,
      W_src_pa, as_pa, ad_pa,
      W_src_pp, as_pp, ad_pp)
    (hl_ap, hh_ap, acs_ap, acd_ap,
     hl_pa, hh_pa, acs_pa, acd_pa,
     hl_pp, hh_pp, acs_pp, acd_pp) = pre_out

    se_ap = edge_ap[0].reshape(NW, NCH, CH)
    de_ap = edge_ap[1].reshape(NW, NCH, CH)
    se_pa = edge_pa[0].reshape(NW, NCH, CH)
    de_pa = edge_pa[1].reshape(NW, NCH, CH)
    se_pp = edge_pp[0].reshape(NW, NCH, CH)
    de_pp = edge_pp[1].reshape(NW, NCH, CH)

    (ol_ap, oh_ap, ol_pa, oh_pa, ol_pp, oh_pp) = _sc_fn(
        hl_ap, hh_ap, acs_ap, acd_ap, se_ap, de_ap,
        hl_pa, hh_pa, acs_pa, acd_pa, se_pa, de_pa,
        hl_pp, hh_pp, acs_pp, acd_pp, se_pp, de_pp)

    grid = NPAD // _BLK
    oa, gap, gpp, ks = pl.pallas_call(
        _post_body,
        grid=(grid,),
        in_specs=[
            pl.BlockSpec((NC, _BLK, AW), lambda i: (0, i, 0)),
            pl.BlockSpec((NC, _BLK, AW), lambda i: (0, i, 0)),
            pl.BlockSpec((NC, _BLK, AW), lambda i: (0, i, 0)),
            pl.BlockSpec((NC, _BLK, AW), lambda i: (0, i, 0)),
            pl.BlockSpec((NC, _BLK, AW), lambda i: (0, i, 0)),
            pl.BlockSpec((NC, _BLK, AW), lambda i: (0, i, 0)),
            pl.BlockSpec((1, D), lambda i: (0, 0)),
            pl.BlockSpec((1, D), lambda i: (0, 0)),
            pl.BlockSpec((1, D), lambda i: (0, 0)),
            pl.BlockSpec((D, D), lambda i: (0, 0)),
            pl.BlockSpec((1, D), lambda i: (0, 0)),
        ],
        out_specs=[
            pl.BlockSpec((_BLK, D), lambda i: (i, 0)),
            pl.BlockSpec((_BLK, D), lambda i: (i, 0)),
            pl.BlockSpec((_BLK, D), lambda i: (i, 0)),
            pl.BlockSpec((2, D), lambda i: (0, 0)),
        ],
        out_shape=[
            jax.ShapeDtypeStruct((N, D), f32),
            jax.ShapeDtypeStruct((N, D), f32),
            jax.ShapeDtypeStruct((N, D), f32),
            jax.ShapeDtypeStruct((2, D), f32),
        ],
    )(ol_ap, oh_ap, ol_pa, oh_pa, ol_pp, oh_pp,
      bias_ap.reshape(1, D), bias_pa.reshape(1, D), bias_pp.reshape(1, D),
      W_k, b_k.reshape(1, D))

    out_paper = pl.pallas_call(
        _fin_body,
        grid=(grid,),
        in_specs=[
            pl.BlockSpec((_BLK, D), lambda i: (i, 0)),
            pl.BlockSpec((_BLK, D), lambda i: (i, 0)),
            pl.BlockSpec((2, D), lambda i: (0, 0)),
            pl.BlockSpec((1, D), lambda i: (0, 0)),
        ],
        out_specs=pl.BlockSpec((_BLK, D), lambda i: (i, 0)),
        out_shape=jax.ShapeDtypeStruct((N, D), f32),
    )(gap, gpp, ks, q)

    return (oa, out_paper)


# speedup vs baseline: 1.2949x; 1.1858x over previous
import jax
import jax.numpy as jnp
from jax import lax
from jax.experimental import pallas as pl
from jax.experimental.pallas import tpu as pltpu
from jax.experimental.pallas import tpu_sc as plsc

N = 10000
HEADS = 8
HID = 16
D = HEADS * HID
HD = D // 2
E = 320000
NEG = 0.2

NC = 2
NS = 16
NW = NC * NS
EPT = E // NW
CH = 80
NCH = EPT // CH
NPAD = 10240
RPT = NPAD // NS
RZ = 128


def _pre_body(xa, xp, wfa, bfa, wfp, bfp,
              ws_ap, as_ap, ad_ap,
              ws_pa, as_pa, ad_pa,
              ws_pp, as_pp, ad_pp,
              hl_ap, hh_ap, acs_ap, acd_ap,
              hl_pa, hh_pa, acs_pa, acd_pa,
              hl_pp, hh_pp, acs_pp, acd_pp):
    f32 = jnp.float32
    ha = jnp.dot(xa[...], wfa[...], preferred_element_type=f32) + bfa[0]
    hp = jnp.dot(xp[...], wfp[...], preferred_element_type=f32) + bfp[0]

    def et(hsrc, hdst, ws, a_s, a_d, hl_o, hh_o, acs_o, acd_o):
        hs = jnp.dot(hsrc, ws[...], preferred_element_type=f32)
        hl_o[...] = hs[:, :HD]
        hh_o[...] = hs[:, HD:]
        acs_o[...] = jnp.dot(hsrc, a_s[...], preferred_element_type=f32)
        acd_o[...] = jnp.dot(hdst, a_d[...], preferred_element_type=f32)

    et(ha, hp, ws_ap, as_ap, ad_ap, hl_ap, hh_ap, acs_ap, acd_ap)
    et(hp, ha, ws_pa, as_pa, ad_pa, hl_pa, hh_pa, acs_pa, acd_pa)
    et(hp, hp, ws_pp, as_pp, ad_pp, hl_pp, hh_pp, acs_pp, acd_pp)


def _sc_body(hl_ap, hh_ap, acs_ap, acd_ap, se_ap, de_ap,
             hl_pa, hh_pa, acs_pa, acd_pa, se_pa, de_pa,
             hl_pp, hh_pp, acs_pp, acd_pp, se_pp, de_pp,
             ol_ap, oh_ap, dp_ap, ol_pa, oh_pa, dp_pa, ol_pp, oh_pp, dp_pp,
             idx_s, idx_d, arow_s, arow_d, w_buf, rows, msg,
             zb_o, zb_d, out_sh, den_sh, gsem):
    c = lax.axis_index("c")
    s = lax.axis_index("s")
    wid = c * NS + s
    f32 = jnp.float32
    zv = jnp.zeros((16,), f32)

    def zrow_o(r, carry):
        for h in range(HD // 16):
            zb_o[r, pl.ds(h * 16, 16)] = zv
        return carry
    lax.fori_loop(0, RZ, zrow_o, 0)

    def zrow_d(r, carry):
        zb_d[r] = zv
        return carry
    lax.fori_loop(0, RZ, zrow_d, 0)

    for (hs_hs, acs_h, acd_h, se_h, de_h, op_hs, dp_h) in (
            ((hl_ap, hh_ap), acs_ap, acd_ap, se_ap, de_ap,
             (ol_ap, oh_ap), dp_ap),
            ((hl_pa, hh_pa), acs_pa, acd_pa, se_pa, de_pa,
             (ol_pa, oh_pa), dp_pa),
            ((hl_pp, hh_pp), acs_pp, acd_pp, se_pp, de_pp,
             (ol_pp, oh_pp), dp_pp)):
        pltpu.sync_copy(se_h.at[wid], idx_s)
        pltpu.sync_copy(de_h.at[wid], idx_d)

        for half in (0, 1):
            hs_h = hs_hs[half]
            op_h = op_hs[half]
            for jz in range(RPT // RZ):
                base = s * RPT + jz * RZ
                pltpu.sync_copy(zb_o, out_sh.at[pl.ds(base, RZ)])
                if half == 0:
                    pltpu.sync_copy(zb_d, den_sh.at[pl.ds(base, RZ)])
            plsc.subcore_barrier()

            def chunk(j, carry):
                si = idx_s.at[j]
                di = idx_d.at[j]
                c1 = pltpu.async_copy(acs_h.at[si], arow_s, gsem)
                c2 = pltpu.async_copy(acd_h.at[di], arow_d, gsem)
                c3 = pltpu.async_copy(hs_h.at[si], rows, gsem)
                c1.wait()
                c2.wait()

                def wcomp(e, cy):
                    x = arow_s[e] + arow_d[e]
                    w_buf[e] = jnp.exp(jnp.maximum(x, NEG * x))
                    return cy
                lax.fori_loop(0, CH, wcomp, 0)
                c3.wait()

                if half == 0:
                    c4 = pltpu.async_copy(w_buf, den_sh.at[di], gsem,
                                          add=True)

                def mcomp(e, cy):
                    wrow = w_buf[e]
                    for h in range(HD // HID):
                        msg[e, pl.ds(h * HID, HID)] = (
                            rows[e, pl.ds(h * HID, HID)]
                            * wrow[half * (HD // HID) + h])
                    return cy
                lax.fori_loop(0, CH, mcomp, 0)

                c5 = pltpu.async_copy(msg, out_sh.at[di], gsem, add=True)
                if half == 0:
                    c4.wait()
                c5.wait()
                return carry
            lax.fori_loop(0, NCH, chunk, 0)

            plsc.subcore_barrier()
            pltpu.sync_copy(out_sh.at[pl.ds(s * RPT, RPT)],
                            op_h.at[c, pl.ds(s * RPT, RPT)])
            if half == 0:
                pltpu.sync_copy(den_sh.at[pl.ds(s * RPT, RPT)],
                                dp_h.at[c, pl.ds(s * RPT, RPT)])
            plsc.subcore_barrier()


_sc_mesh = plsc.VectorSubcoreMesh(
    core_axis_name="c", subcore_axis_name="s", num_cores=NC, num_subcores=NS)

_sc_fn = pl.kernel(
    _sc_body,
    out_type=[
        jax.ShapeDtypeStruct((NC, NPAD, HD), jnp.float32),
        jax.ShapeDtypeStruct((NC, NPAD, HD), jnp.float32),
        jax.ShapeDtypeStruct((NC, NPAD, 16), jnp.float32),
        jax.ShapeDtypeStruct((NC, NPAD, HD), jnp.float32),
        jax.ShapeDtypeStruct((NC, NPAD, HD), jnp.float32),
        jax.ShapeDtypeStruct((NC, NPAD, 16), jnp.float32),
        jax.ShapeDtypeStruct((NC, NPAD, HD), jnp.float32),
        jax.ShapeDtypeStruct((NC, NPAD, HD), jnp.float32),
        jax.ShapeDtypeStruct((NC, NPAD, 16), jnp.float32),
    ],
    mesh=_sc_mesh,
    compiler_params=pltpu.CompilerParams(use_tc_tiling_on_sc=False),
    scratch_types=[
        pltpu.VMEM((NCH, CH), jnp.int32),
        pltpu.VMEM((NCH, CH), jnp.int32),
        pltpu.VMEM((CH, 16), jnp.float32),
        pltpu.VMEM((CH, 16), jnp.float32),
        pltpu.VMEM((CH, 16), jnp.float32),
        pltpu.VMEM((CH, HD), jnp.float32),
        pltpu.VMEM((CH, HD), jnp.float32),
        pltpu.VMEM((RZ, HD), jnp.float32),
        pltpu.VMEM((RZ, 16), jnp.float32),
        pltpu.VMEM_SHARED((NPAD, HD), jnp.float32),
        pltpu.VMEM_SHARED((NPAD, 16), jnp.float32),
        pltpu.SemaphoreType.DMA,
    ],
)


_BLK = 1280


def _post_body(ol_ap, oh_ap, dp_ap, ol_pa, oh_pa, dp_pa, ol_pp, oh_pp, dp_pp,
               b_ap, b_pa, b_pp, wk, bk,
               oa_ref, gap_ref, gpp_ref, ks_ref):
    i = pl.program_id(0)
    f32 = jnp.float32

    def gat(ol, oh, dp, b):
        o = jnp.concatenate([ol[0] + ol[1], oh[0] + oh[1]], axis=1)
        d = (dp[0] + dp[1])[:, :HEADS]
        rj = lax.broadcasted_iota(jnp.int32, (HEADS, D), 0)
        rl = lax.broadcasted_iota(jnp.int32, (HEADS, D), 1)
        p = (rl // HID == rj).astype(f32)
        dex = jnp.dot(d, p, preferred_element_type=f32)
        return o / jnp.maximum(dex, 1e-16) + b[0]

    g_pa = gat(ol_pa, oh_pa, dp_pa, b_pa)
    oa_ref[...] = g_pa
    g_ap = gat(ol_ap, oh_ap, dp_ap, b_ap)
    gap_ref[...] = g_ap
    g_pp = gat(ol_pp, oh_pp, dp_pp, b_pp)
    gpp_ref[...] = g_pp

    row = (i * _BLK
           + lax.broadcasted_iota(jnp.int32, (_BLK, 1), 0))
    valid = (row < N).astype(f32)
    k_ap = jnp.sum(valid * jnp.tanh(
        jnp.dot(g_ap, wk[...], preferred_element_type=f32) + bk[0]),
        axis=0, keepdims=True)
    k_pp = jnp.sum(valid * jnp.tanh(
        jnp.dot(g_pp, wk[...], preferred_element_type=f32) + bk[0]),
        axis=0, keepdims=True)
    kb = jnp.concatenate([k_ap, k_pp], axis=0)

    @pl.when(i == 0)
    def _():
        ks_ref[...] = kb

    @pl.when(i != 0)
    def _():
        ks_ref[...] = ks_ref[...] + kb


def _fin_body(gap, gpp, ks, q, op_ref):
    kq = ks[...] * q[...]
    a0 = jnp.sum(kq[0]) / N
    a1 = jnp.sum(kq[1]) / N
    m = jnp.maximum(a0, a1)
    e0 = jnp.exp(a0 - m)
    e1 = jnp.exp(a1 - m)
    s0 = e0 / (e0 + e1)
    s1 = e1 / (e0 + e1)
    op_ref[...] = s0 * gap[...] + s1 * gpp[...]


def kernel(x_author, x_paper, edge_ap, edge_pa, edge_pp,
           W_fc_author, b_fc_author, W_fc_paper, b_fc_paper,
           W_src_ap, W_dst_ap, att_src_ap, att_dst_ap, bias_ap,
           W_src_pa, W_dst_pa, att_src_pa, att_dst_pa, bias_pa,
           W_src_pp, W_dst_pp, att_src_pp, att_dst_pp, bias_pp,
           q, W_k, b_k):
    f32 = jnp.float32

    def fold(w_src, att_src, w_dst, att_dst):
        a_s = (w_src.reshape(HID, HEADS, HID)
               * att_src.reshape(1, HEADS, HID)).sum(-1)
        a_d = (w_dst.reshape(HID, HEADS, HID)
               * att_dst.reshape(1, HEADS, HID)).sum(-1)
        z = jnp.zeros((HID, HEADS), f32)
        return (jnp.concatenate([a_s, z], axis=1),
                jnp.concatenate([a_d, z], axis=1))

    as_ap, ad_ap = fold(W_src_ap, att_src_ap, W_dst_ap, att_dst_ap)
    as_pa, ad_pa = fold(W_src_pa, att_src_pa, W_dst_pa, att_dst_pa)
    as_pp, ad_pp = fold(W_src_pp, att_src_pp, W_dst_pp, att_dst_pp)

    pb = 2000
    pre_out = pl.pallas_call(
        _pre_body,
        grid=(N // pb,),
        in_specs=[
            pl.BlockSpec((pb, D), lambda i: (i, 0)),
            pl.BlockSpec((pb, D), lambda i: (i, 0)),
            pl.BlockSpec((D, HID), lambda i: (0, 0)),
            pl.BlockSpec((1, HID), lambda i: (0, 0)),
            pl.BlockSpec((D, HID), lambda i: (0, 0)),
            pl.BlockSpec((1, HID), lambda i: (0, 0)),
        ] + [
            pl.BlockSpec((HID, D), lambda i: (0, 0)),
            pl.BlockSpec((HID, HID), lambda i: (0, 0)),
            pl.BlockSpec((HID, HID), lambda i: (0, 0)),
        ] * 3,
        out_specs=[
            pl.BlockSpec((pb, HD), lambda i: (i, 0)),
            pl.BlockSpec((pb, HD), lambda i: (i, 0)),
            pl.BlockSpec((pb, 16), lambda i: (i, 0)),
            pl.BlockSpec((pb, 16), lambda i: (i, 0)),
        ] * 3,
        out_shape=[
            jax.ShapeDtypeStruct((N, HD), f32),
            jax.ShapeDtypeStruct((N, HD), f32),
            jax.ShapeDtypeStruct((N, 16), f32),
            jax.ShapeDtypeStruct((N, 16), f32),
        ] * 3,
    )(x_author, x_paper,
      W_fc_author, b_fc_author.reshape(1, HID),
      W_fc_paper, b_fc_paper.reshape(1, HID),
      W_src_ap, as_ap, ad_ap,
      W_src_pa, as_pa, ad_pa,
      W_src_pp, as_pp, ad_pp)
    (hl_ap, hh_ap, acs_ap, acd_ap,
     hl_pa, hh_pa, acs_pa, acd_pa,
     hl_pp, hh_pp, acs_pp, acd_pp) = pre_out

    se_ap = edge_ap[0].reshape(NW, NCH, CH)
    de_ap = edge_ap[1].reshape(NW, NCH, CH)
    se_pa = edge_pa[0].reshape(NW, NCH, CH)
    de_pa = edge_pa[1].reshape(NW, NCH, CH)
    se_pp = edge_pp[0].reshape(NW, NCH, CH)
    de_pp = edge_pp[1].reshape(NW, NCH, CH)

    (ol_ap, oh_ap, dp_ap, ol_pa, oh_pa, dp_pa,
     ol_pp, oh_pp, dp_pp) = _sc_fn(
        hl_ap, hh_ap, acs_ap, acd_ap, se_ap, de_ap,
        hl_pa, hh_pa, acs_pa, acd_pa, se_pa, de_pa,
        hl_pp, hh_pp, acs_pp, acd_pp, se_pp, de_pp)

    grid = NPAD // _BLK
    oa, gap, gpp, ks = pl.pallas_call(
        _post_body,
        grid=(grid,),
        in_specs=[
            pl.BlockSpec((NC, _BLK, HD), lambda i: (0, i, 0)),
            pl.BlockSpec((NC, _BLK, HD), lambda i: (0, i, 0)),
            pl.BlockSpec((NC, _BLK, 16), lambda i: (0, i, 0)),
            pl.BlockSpec((NC, _BLK, HD), lambda i: (0, i, 0)),
            pl.BlockSpec((NC, _BLK, HD), lambda i: (0, i, 0)),
            pl.BlockSpec((NC, _BLK, 16), lambda i: (0, i, 0)),
            pl.BlockSpec((NC, _BLK, HD), lambda i: (0, i, 0)),
            pl.BlockSpec((NC, _BLK, HD), lambda i: (0, i, 0)),
            pl.BlockSpec((NC, _BLK, 16), lambda i: (0, i, 0)),
            pl.BlockSpec((1, D), lambda i: (0, 0)),
            pl.BlockSpec((1, D), lambda i: (0, 0)),
            pl.BlockSpec((1, D), lambda i: (0, 0)),
            pl.BlockSpec((D, D), lambda i: (0, 0)),
            pl.BlockSpec((1, D), lambda i: (0, 0)),
        ],
        out_specs=[
            pl.BlockSpec((_BLK, D), lambda i: (i, 0)),
            pl.BlockSpec((_BLK, D), lambda i: (i, 0)),
            pl.BlockSpec((_BLK, D), lambda i: (i, 0)),
            pl.BlockSpec((2, D), lambda i: (0, 0)),
        ],
        out_shape=[
            jax.ShapeDtypeStruct((N, D), f32),
            jax.ShapeDtypeStruct((N, D), f32),
            jax.ShapeDtypeStruct((N, D), f32),
            jax.ShapeDtypeStruct((2, D), f32),
        ],
    )(ol_ap, oh_ap, dp_ap, ol_pa, oh_pa, dp_pa, ol_pp, oh_pp, dp_pp,
      bias_ap.reshape(1, D), bias_pa.reshape(1, D), bias_pp.reshape(1, D),
      W_k, b_k.reshape(1, D))

    out_paper = pl.pallas_call(
        _fin_body,
        grid=(grid,),
        in_specs=[
            pl.BlockSpec((_BLK, D), lambda i: (i, 0)),
            pl.BlockSpec((_BLK, D), lambda i: (i, 0)),
            pl.BlockSpec((2, D), lambda i: (0, 0)),
            pl.BlockSpec((1, D), lambda i: (0, 0)),
        ],
        out_specs=pl.BlockSpec((_BLK, D), lambda i: (i, 0)),
        out_shape=jax.ShapeDtypeStruct((N, D), f32),
    )(gap, gpp, ks, q)

    return (oa, out_paper)


# speedup vs baseline: 1.4310x; 1.1051x over previous
import jax
import jax.numpy as jnp
from jax import lax
from jax.experimental import pallas as pl
from jax.experimental.pallas import tpu as pltpu
from jax.experimental.pallas import tpu_sc as plsc

N = 10000
HEADS = 8
HID = 16
D = HEADS * HID
HD = D // 2
E = 320000
NEG = 0.2

NC = 2
NS = 16
NW = NC * NS
EPT = E // NW
CH = 80
NCH = EPT // CH
NPAD = 10240
RPT = NPAD // NS
RZ = 128


def _pre_body(xa, xp, wfa, bfa, wfp, bfp,
              ws_ap, as_ap, ad_ap,
              ws_pa, as_pa, ad_pa,
              ws_pp, as_pp, ad_pp,
              hl_ap, hh_ap, acs_ap, acd_ap,
              hl_pa, hh_pa, acs_pa, acd_pa,
              hl_pp, hh_pp, acs_pp, acd_pp):
    f32 = jnp.float32
    ha = jnp.dot(xa[...], wfa[...], preferred_element_type=f32) + bfa[0]
    hp = jnp.dot(xp[...], wfp[...], preferred_element_type=f32) + bfp[0]

    def et(hsrc, hdst, ws, a_s, a_d, hl_o, hh_o, acs_o, acd_o):
        hs = jnp.dot(hsrc, ws[...], preferred_element_type=f32)
        hl_o[...] = hs[:, :HD]
        hh_o[...] = hs[:, HD:]
        acs_o[...] = jnp.dot(hsrc, a_s[...], preferred_element_type=f32)
        acd_o[...] = jnp.dot(hdst, a_d[...], preferred_element_type=f32)

    et(ha, hp, ws_ap, as_ap, ad_ap, hl_ap, hh_ap, acs_ap, acd_ap)
    et(hp, ha, ws_pa, as_pa, ad_pa, hl_pa, hh_pa, acs_pa, acd_pa)
    et(hp, hp, ws_pp, as_pp, ad_pp, hl_pp, hh_pp, acs_pp, acd_pp)


def _sc_body(hl_ap, hh_ap, acs_ap, acd_ap, se_ap, de_ap,
             hl_pa, hh_pa, acs_pa, acd_pa, se_pa, de_pa,
             hl_pp, hh_pp, acs_pp, acd_pp, se_pp, de_pp,
             ol_ap, oh_ap, dp_ap, ol_pa, oh_pa, dp_pa, ol_pp, oh_pp, dp_pp,
             idx_s, idx_d, arow_s, arow_d, w_buf, rows, msg,
             zb_o, zb_d, out_sh, den_sh, gsem, ssem):
    c = lax.axis_index("c")
    s = lax.axis_index("s")
    wid = c * NS + s
    f32 = jnp.float32
    zv = jnp.zeros((16,), f32)

    def zrow_o(r, carry):
        for h in range(HD // 16):
            zb_o[r, pl.ds(h * 16, 16)] = zv
        return carry
    lax.fori_loop(0, RZ, zrow_o, 0)

    def zrow_d(r, carry):
        zb_d[r] = zv
        return carry
    lax.fori_loop(0, RZ, zrow_d, 0)

    for (hs_hs, acs_h, acd_h, se_h, de_h, op_hs, dp_h) in (
            ((hl_ap, hh_ap), acs_ap, acd_ap, se_ap, de_ap,
             (ol_ap, oh_ap), dp_ap),
            ((hl_pa, hh_pa), acs_pa, acd_pa, se_pa, de_pa,
             (ol_pa, oh_pa), dp_pa),
            ((hl_pp, hh_pp), acs_pp, acd_pp, se_pp, de_pp,
             (ol_pp, oh_pp), dp_pp)):
        pltpu.sync_copy(se_h.at[wid], idx_s)
        pltpu.sync_copy(de_h.at[wid], idx_d)

        for half in (0, 1):
            hs_h = hs_hs[half]
            op_h = op_hs[half]
            for jz in range(RPT // RZ):
                base = s * RPT + jz * RZ
                pltpu.sync_copy(zb_o, out_sh.at[pl.ds(base, RZ)])
                if half == 0:
                    pltpu.sync_copy(zb_d, den_sh.at[pl.ds(base, RZ)])
            plsc.subcore_barrier()

            def zmsg(e, carry):
                for h in range(HD // HID):
                    msg[e, pl.ds(h * HID, HID)] = zv
                return carry
            lax.fori_loop(0, CH, zmsg, 0)
            pltpu.async_copy(msg, out_sh.at[idx_d.at[0]], ssem, add=True)

            def chunk(j, carry):
                si = idx_s.at[j]
                di = idx_d.at[j]
                c1 = pltpu.async_copy(acs_h.at[si], arow_s, gsem)
                c2 = pltpu.async_copy(acd_h.at[di], arow_d, gsem)
                c3 = pltpu.async_copy(hs_h.at[si], rows, gsem)
                c1.wait()
                c2.wait()

                def wcomp(e, cy):
                    x = arow_s[e] + arow_d[e]
                    w_buf[e] = jnp.exp(jnp.maximum(x, NEG * x))
                    return cy
                lax.fori_loop(0, CH, wcomp, 0)
                c3.wait()
                pltpu.make_async_copy(msg, out_sh.at[di], ssem).wait()

                if half == 0:
                    c4 = pltpu.async_copy(w_buf, den_sh.at[di], gsem,
                                          add=True)

                def mcomp(e, cy):
                    wrow = w_buf[e]
                    for h in range(HD // HID):
                        msg[e, pl.ds(h * HID, HID)] = (
                            rows[e, pl.ds(h * HID, HID)]
                            * wrow[half * (HD // HID) + h])
                    return cy
                lax.fori_loop(0, CH, mcomp, 0)

                pltpu.async_copy(msg, out_sh.at[di], ssem, add=True)
                if half == 0:
                    c4.wait()
                return carry
            lax.fori_loop(0, NCH, chunk, 0)

            pltpu.make_async_copy(msg, out_sh.at[idx_d.at[0]], ssem).wait()

            plsc.subcore_barrier()
            pltpu.sync_copy(out_sh.at[pl.ds(s * RPT, RPT)],
                            op_h.at[c, pl.ds(s * RPT, RPT)])
            if half == 0:
                pltpu.sync_copy(den_sh.at[pl.ds(s * RPT, RPT)],
                                dp_h.at[c, pl.ds(s * RPT, RPT)])
            plsc.subcore_barrier()


_sc_mesh = plsc.VectorSubcoreMesh(
    core_axis_name="c", subcore_axis_name="s", num_cores=NC, num_subcores=NS)

_sc_fn = pl.kernel(
    _sc_body,
    out_type=[
        jax.ShapeDtypeStruct((NC, NPAD, HD), jnp.float32),
        jax.ShapeDtypeStruct((NC, NPAD, HD), jnp.float32),
        jax.ShapeDtypeStruct((NC, NPAD, 16), jnp.float32),
        jax.ShapeDtypeStruct((NC, NPAD, HD), jnp.float32),
        jax.ShapeDtypeStruct((NC, NPAD, HD), jnp.float32),
        jax.ShapeDtypeStruct((NC, NPAD, 16), jnp.float32),
        jax.ShapeDtypeStruct((NC, NPAD, HD), jnp.float32),
        jax.ShapeDtypeStruct((NC, NPAD, HD), jnp.float32),
        jax.ShapeDtypeStruct((NC, NPAD, 16), jnp.float32),
    ],
    mesh=_sc_mesh,
    compiler_params=pltpu.CompilerParams(use_tc_tiling_on_sc=False),
    scratch_types=[
        pltpu.VMEM((NCH, CH), jnp.int32),
        pltpu.VMEM((NCH, CH), jnp.int32),
        pltpu.VMEM((CH, 16), jnp.float32),
        pltpu.VMEM((CH, 16), jnp.float32),
        pltpu.VMEM((CH, 16), jnp.float32),
        pltpu.VMEM((CH, HD), jnp.float32),
        pltpu.VMEM((CH, HD), jnp.float32),
        pltpu.VMEM((RZ, HD), jnp.float32),
        pltpu.VMEM((RZ, 16), jnp.float32),
        pltpu.VMEM_SHARED((NPAD, HD), jnp.float32),
        pltpu.VMEM_SHARED((NPAD, 16), jnp.float32),
        pltpu.SemaphoreType.DMA,
        pltpu.SemaphoreType.DMA,
    ],
)


_BLK = 1280


def _post_body(ol_ap, oh_ap, dp_ap, ol_pa, oh_pa, dp_pa, ol_pp, oh_pp, dp_pp,
               b_ap, b_pa, b_pp, wk, bk,
               oa_ref, gap_ref, gpp_ref, ks_ref):
    i = pl.program_id(0)
    f32 = jnp.float32

    def gat(ol, oh, dp, b):
        o = jnp.concatenate([ol[0] + ol[1], oh[0] + oh[1]], axis=1)
        d = (dp[0] + dp[1])[:, :HEADS]
        rj = lax.broadcasted_iota(jnp.int32, (HEADS, D), 0)
        rl = lax.broadcasted_iota(jnp.int32, (HEADS, D), 1)
        p = (rl // HID == rj).astype(f32)
        dex = jnp.dot(d, p, preferred_element_type=f32)
        return o / jnp.maximum(dex, 1e-16) + b[0]

    g_pa = gat(ol_pa, oh_pa, dp_pa, b_pa)
    oa_ref[...] = g_pa
    g_ap = gat(ol_ap, oh_ap, dp_ap, b_ap)
    gap_ref[...] = g_ap
    g_pp = gat(ol_pp, oh_pp, dp_pp, b_pp)
    gpp_ref[...] = g_pp

    row = (i * _BLK
           + lax.broadcasted_iota(jnp.int32, (_BLK, 1), 0))
    valid = (row < N).astype(f32)
    k_ap = jnp.sum(valid * jnp.tanh(
        jnp.dot(g_ap, wk[...], preferred_element_type=f32) + bk[0]),
        axis=0, keepdims=True)
    k_pp = jnp.sum(valid * jnp.tanh(
        jnp.dot(g_pp, wk[...], preferred_element_type=f32) + bk[0]),
        axis=0, keepdims=True)
    kb = jnp.concatenate([k_ap, k_pp], axis=0)

    @pl.when(i == 0)
    def _():
        ks_ref[...] = kb

    @pl.when(i != 0)
    def _():
        ks_ref[...] = ks_ref[...] + kb


def _fin_body(gap, gpp, ks, q, op_ref):
    kq = ks[...] * q[...]
    a0 = jnp.sum(kq[0]) / N
    a1 = jnp.sum(kq[1]) / N
    m = jnp.maximum(a0, a1)
    e0 = jnp.exp(a0 - m)
    e1 = jnp.exp(a1 - m)
    s0 = e0 / (e0 + e1)
    s1 = e1 / (e0 + e1)
    op_ref[...] = s0 * gap[...] + s1 * gpp[...]


def kernel(x_author, x_paper, edge_ap, edge_pa, edge_pp,
           W_fc_author, b_fc_author, W_fc_paper, b_fc_paper,
           W_src_ap, W_dst_ap, att_src_ap, att_dst_ap, bias_ap,
           W_src_pa, W_dst_pa, att_src_pa, att_dst_pa, bias_pa,
           W_src_pp, W_dst_pp, att_src_pp, att_dst_pp, bias_pp,
           q, W_k, b_k):
    f32 = jnp.float32

    def fold(w_src, att_src, w_dst, att_dst):
        a_s = (w_src.reshape(HID, HEADS, HID)
               * att_src.reshape(1, HEADS, HID)).sum(-1)
        a_d = (w_dst.reshape(HID, HEADS, HID)
               * att_dst.reshape(1, HEADS, HID)).sum(-1)
        z = jnp.zeros((HID, HEADS), f32)
        return (jnp.concatenate([a_s, z], axis=1),
                jnp.concatenate([a_d, z], axis=1))

    as_ap, ad_ap = fold(W_src_ap, att_src_ap, W_dst_ap, att_dst_ap)
    as_pa, ad_pa = fold(W_src_pa, att_src_pa, W_dst_pa, att_dst_pa)
    as_pp, ad_pp = fold(W_src_pp, att_src_pp, W_dst_pp, att_dst_pp)

    pb = 2000
    pre_out = pl.pallas_call(
        _pre_body,
        grid=(N // pb,),
        in_specs=[
            pl.BlockSpec((pb, D), lambda i: (i, 0)),
            pl.BlockSpec((pb, D), lambda i: (i, 0)),
            pl.BlockSpec((D, HID), lambda i: (0, 0)),
            pl.BlockSpec((1, HID), lambda i: (0, 0)),
            pl.BlockSpec((D, HID), lambda i: (0, 0)),
            pl.BlockSpec((1, HID), lambda i: (0, 0)),
        ] + [
            pl.BlockSpec((HID, D), lambda i: (0, 0)),
            pl.BlockSpec((HID, HID), lambda i: (0, 0)),
            pl.BlockSpec((HID, HID), lambda i: (0, 0)),
        ] * 3,
        out_specs=[
            pl.BlockSpec((pb, HD), lambda i: (i, 0)),
            pl.BlockSpec((pb, HD), lambda i: (i, 0)),
            pl.BlockSpec((pb, 16), lambda i: (i, 0)),
            pl.BlockSpec((pb, 16), lambda i: (i, 0)),
        ] * 3,
        out_shape=[
            jax.ShapeDtypeStruct((N, HD), f32),
            jax.ShapeDtypeStruct((N, HD), f32),
            jax.ShapeDtypeStruct((N, 16), f32),
            jax.ShapeDtypeStruct((N, 16), f32),
        ] * 3,
    )(x_author, x_paper,
      W_fc_author, b_fc_author.reshape(1, HID),
      W_fc_paper, b_fc_paper.reshape(1, HID),
      W_src_ap, as_ap, ad_ap,
      W_src_pa, as_pa, ad_pa,
      W_src_pp, as_pp, ad_pp)
    (hl_ap, hh_ap, acs_ap, acd_ap,
     hl_pa, hh_pa, acs_pa, acd_pa,
     hl_pp, hh_pp, acs_pp, acd_pp) = pre_out

    se_ap = edge_ap[0].reshape(NW, NCH, CH)
    de_ap = edge_ap[1].reshape(NW, NCH, CH)
    se_pa = edge_pa[0].reshape(NW, NCH, CH)
    de_pa = edge_pa[1].reshape(NW, NCH, CH)
    se_pp = edge_pp[0].reshape(NW, NCH, CH)
    de_pp = edge_pp[1].reshape(NW, NCH, CH)

    (ol_ap, oh_ap, dp_ap, ol_pa, oh_pa, dp_pa,
     ol_pp, oh_pp, dp_pp) = _sc_fn(
        hl_ap, hh_ap, acs_ap, acd_ap, se_ap, de_ap,
        hl_pa, hh_pa, acs_pa, acd_pa, se_pa, de_pa,
        hl_pp, hh_pp, acs_pp, acd_pp, se_pp, de_pp)

    grid = NPAD // _BLK
    oa, gap, gpp, ks = pl.pallas_call(
        _post_body,
        grid=(grid,),
        in_specs=[
            pl.BlockSpec((NC, _BLK, HD), lambda i: (0, i, 0)),
            pl.BlockSpec((NC, _BLK, HD), lambda i: (0, i, 0)),
            pl.BlockSpec((NC, _BLK, 16), lambda i: (0, i, 0)),
            pl.BlockSpec((NC, _BLK, HD), lambda i: (0, i, 0)),
            pl.BlockSpec((NC, _BLK, HD), lambda i: (0, i, 0)),
            pl.BlockSpec((NC, _BLK, 16), lambda i: (0, i, 0)),
            pl.BlockSpec((NC, _BLK, HD), lambda i: (0, i, 0)),
            pl.BlockSpec((NC, _BLK, HD), lambda i: (0, i, 0)),
            pl.BlockSpec((NC, _BLK, 16), lambda i: (0, i, 0)),
            pl.BlockSpec((1, D), lambda i: (0, 0)),
            pl.BlockSpec((1, D), lambda i: (0, 0)),
            pl.BlockSpec((1, D), lambda i: (0, 0)),
            pl.BlockSpec((D, D), lambda i: (0, 0)),
            pl.BlockSpec((1, D), lambda i: (0, 0)),
        ],
        out_specs=[
            pl.BlockSpec((_BLK, D), lambda i: (i, 0)),
            pl.BlockSpec((_BLK, D), lambda i: (i, 0)),
            pl.BlockSpec((_BLK, D), lambda i: (i, 0)),
            pl.BlockSpec((2, D), lambda i: (0, 0)),
        ],
        out_shape=[
            jax.ShapeDtypeStruct((N, D), f32),
            jax.ShapeDtypeStruct((N, D), f32),
            jax.ShapeDtypeStruct((N, D), f32),
            jax.ShapeDtypeStruct((2, D), f32),
        ],
    )(ol_ap, oh_ap, dp_ap, ol_pa, oh_pa, dp_pa, ol_pp, oh_pp, dp_pp,
      bias_ap.reshape(1, D), bias_pa.reshape(1, D), bias_pp.reshape(1, D),
      W_k, b_k.reshape(1, D))

    out_paper = pl.pallas_call(
        _fin_body,
        grid=(grid,),
        in_specs=[
            pl.BlockSpec((_BLK, D), lambda i: (i, 0)),
            pl.BlockSpec((_BLK, D), lambda i: (i, 0)),
            pl.BlockSpec((2, D), lambda i: (0, 0)),
            pl.BlockSpec((1, D), lambda i: (0, 0)),
        ],
        out_specs=pl.BlockSpec((_BLK, D), lambda i: (i, 0)),
        out_shape=jax.ShapeDtypeStruct((N, D), f32),
    )(gap, gpp, ks, q)

    return (oa, out_paper)
